# baseline (speedup 1.0000x reference)
import jax
import jax.numpy as jnp
from jax.experimental import pallas as pl
from jax.experimental.pallas import tpu as pltpu

_MAX_TILE_L = 4096
_VMEM_LIMIT_BYTES = 64 * 1024 * 1024


def _csa_kernel(x_ref, o_ref):
    data = x_ref[0]
    logits = x_ref[1]
    m = jnp.max(logits, axis=0, keepdims=True)
    e = jnp.exp(logits - m)
    r = 1.0 / jnp.sum(e, axis=0, keepdims=True)
    o_ref[...] = data * (e * r)


def kernel(x):
    B, C, L = x.shape
    assert C % 2 == 0
    half = C // 2

    x4 = x.reshape(B, 2, half, L)

    if L <= 128:
        tile_l = L
    else:
        tile_l = min(_MAX_TILE_L, (L + 127) // 128 * 128)
    grid_l = pl.cdiv(L, tile_l)

    grid_spec = pl.GridSpec(
        grid=(B, grid_l),
        in_specs=[
            pl.BlockSpec((pl.Squeezed(), 2, half, tile_l),
                         lambda b, l: (b, 0, 0, l)),
        ],
        out_specs=pl.BlockSpec((pl.Squeezed(), half, tile_l),
                               lambda b, l: (b, 0, l)),
    )

    return pl.pallas_call(
        _csa_kernel,
        out_shape=jax.ShapeDtypeStruct((B, half, L), x.dtype),
        grid_spec=grid_spec,
        compiler_params=pltpu.CompilerParams(
            dimension_semantics=("parallel", "parallel"),
            vmem_limit_bytes=_VMEM_LIMIT_BYTES,
        ),
    )(x4)


# speedup vs baseline: 1.0438x; 1.0438x over previous
import jax
import jax.numpy as jnp
from jax.experimental import pallas as pl
from jax.experimental.pallas import tpu as pltpu

_MAX_TILE_L = 4096
_VMEM_LIMIT_BYTES = 64 * 1024 * 1024


def _csa_kernel(x_ref, o_ref):
    data = x_ref[:, 0]
    logits = x_ref[:, 1]
    m = jnp.max(logits, axis=1, keepdims=True)
    e = jnp.exp(logits - m)
    r = 1.0 / jnp.sum(e, axis=1, keepdims=True)
    o_ref[...] = data * (e * r)


def kernel(x):
    B, C, L = x.shape
    assert C % 2 == 0
    half = C // 2

    x4 = x.reshape(B, 2, half, L)

    if L <= 128:
        tile_l = L
    else:
        tile_l = min(_MAX_TILE_L, (L + 127) // 128 * 128)
    grid_l = pl.cdiv(L, tile_l)
    tile_b = 2 if B % 2 == 0 else 1
    grid_b = B // tile_b

    grid_spec = pl.GridSpec(
        grid=(grid_b, grid_l),
        in_specs=[
            pl.BlockSpec((tile_b, 2, half, tile_l),
                         lambda b, l: (b, 0, 0, l)),
        ],
        out_specs=pl.BlockSpec((tile_b, half, tile_l),
                               lambda b, l: (b, 0, l)),
    )

    return pl.pallas_call(
        _csa_kernel,
        out_shape=jax.ShapeDtypeStruct((B, half, L), x.dtype),
        grid_spec=grid_spec,
        compiler_params=pltpu.CompilerParams(
            dimension_semantics=("parallel", "parallel"),
            vmem_limit_bytes=_VMEM_LIMIT_BYTES,
        ),
    )(x4)


# speedup vs baseline: 1.0450x; 1.0012x over previous
import jax
import jax.numpy as jnp
from jax.experimental import pallas as pl
from jax.experimental.pallas import tpu as pltpu

_MAX_TILE_L = 4096
_VMEM_LIMIT_BYTES = 64 * 1024 * 1024


def _csa_kernel(x_ref, o_ref):
    data = x_ref[:, 0]
    logits = x_ref[:, 1]
    m = jnp.max(logits, axis=1, keepdims=True)
    e = jnp.exp(logits - m)
    r = 1.0 / jnp.sum(e, axis=1, keepdims=True)
    o_ref[...] = data * (e * r)


def kernel(x):
    B, C, L = x.shape
    assert C % 2 == 0
    half = C // 2

    x4 = x.reshape(B, 2, half, L)

    if L <= 128:
        tile_l = L
    else:
        tile_l = min(_MAX_TILE_L, (L + 127) // 128 * 128)
    if L > 2048:
        tile_l = 2048
    grid_l = pl.cdiv(L, tile_l)
    tile_b = 4 if B % 4 == 0 else (2 if B % 2 == 0 else 1)
    grid_b = B // tile_b

    grid_spec = pl.GridSpec(
        grid=(grid_b, grid_l),
        in_specs=[
            pl.BlockSpec((tile_b, 2, half, tile_l),
                         lambda b, l: (b, 0, 0, l)),
        ],
        out_specs=pl.BlockSpec((tile_b, half, tile_l),
                               lambda b, l: (b, 0, l)),
    )

    return pl.pallas_call(
        _csa_kernel,
        out_shape=jax.ShapeDtypeStruct((B, half, L), x.dtype),
        grid_spec=grid_spec,
        compiler_params=pltpu.CompilerParams(
            dimension_semantics=("parallel", "parallel"),
            vmem_limit_bytes=_VMEM_LIMIT_BYTES,
        ),
    )(x4)
